# baseline (speedup 1.0000x reference)
import functools

import jax
import jax.numpy as jnp
from jax import lax
from jax.experimental import pallas as pl
from jax.experimental.pallas import tpu as pltpu
from jax.experimental.pallas import tpu_sc as plsc

P31 = (1 << 31) - 1
ROWS_K = 1000000
DIM_K = 32
NCH_K = 8
BATCH_K = 16384
NW = 32
BPW = BATCH_K // NW
CH = 16
NSTEP = BPW // CH
G = CH * NCH_K


def _fold1(v):
    return (v >> jnp.uint32(31)) + (v & jnp.uint32(P31))


def _fold2(v):
    return _fold1(_fold1(v))


def _hash_mod(x0, x1, a0, a1, b, d, inv_d):
    p00 = x0 * a0
    p01 = x0 * a1
    p10 = x1 * a0
    p11 = x1 * a1
    mid = _fold1(p01 + p10)
    t = (mid >> jnp.uint32(15)) + ((mid & jnp.uint32(0x7FFF)) << jnp.uint32(16))
    s = _fold1(_fold2(p00) + _fold2(t))
    s = s + (p11 << jnp.uint32(1)) + b
    h = _fold2(s)
    h = jnp.where(h == jnp.uint32(P31), jnp.uint32(0), h)
    hf = plsc.bitcast(h, jnp.int32).astype(jnp.float32)
    q = (hf * inv_d).astype(jnp.int32)
    r = h - plsc.bitcast(q, jnp.uint32) * jnp.uint32(d)
    r = jnp.where(plsc.bitcast(r, jnp.int32) < 0, r + jnp.uint32(d), r)
    r = jnp.where(r >= jnp.uint32(d), r - jnp.uint32(d), r)
    return plsc.bitcast(r, jnp.int32)


def _emb_body(x_ref, tab_ref, w_ref, c_ref, out_ref,
              x_v, c_v, idx0_v, idx1_v, rows_v, wv_v, outc_v, sem0, sem1):
    cid = lax.axis_index("c")
    sid = lax.axis_index("s")
    wid = sid * jnp.int32(2) + cid
    base = pl.multiple_of(wid * jnp.int32(BPW), BPW)
    pltpu.sync_copy(x_ref.at[pl.ds(base, BPW)], x_v)
    pltpu.sync_copy(c_ref, c_v)

    cv = [c_v[pl.ds(16 * k, 16)] for k in range(4)]

    def cget(i):
        return cv[i // 16][i % 16].astype(jnp.uint32)

    c0 = [(cget(j), cget(8 + j), cget(16 + j)) for j in range(NCH_K)]
    c1 = [(cget(24 + j), cget(32 + j), cget(40 + j)) for j in range(NCH_K)]
    inv0 = jnp.float32(1.0 / ROWS_K)
    inv1 = jnp.float32(1.0 / (ROWS_K * DIM_K))
    iota16 = lax.iota(jnp.int32, 16)

    def step(st, carry):
        off = pl.multiple_of(st * jnp.int32(CH), CH)
        xu = plsc.bitcast(x_v[pl.ds(off, CH)], jnp.uint32)
        x0 = xu & jnp.uint32(0xFFFF)
        x1 = xu >> jnp.uint32(16)
        for j in range(NCH_K):
            a0, a1, b = c0[j]
            idx0_v[pl.ds(j * CH, CH)] = _hash_mod(x0, x1, a0, a1, b,
                                                  ROWS_K, inv0)
            a0, a1, b = c1[j]
            idx1_v[pl.ds(j * CH, CH)] = _hash_mod(x0, x1, a0, a1, b,
                                                  ROWS_K * DIM_K, inv1)
        cp0 = pltpu.async_copy(tab_ref.at[idx0_v], rows_v, sem0)
        cp1 = pltpu.async_copy(w_ref.at[idx1_v], wv_v, sem1)
        cp0.wait()
        cp1.wait()
        wvecs = [wv_v[pl.ds(j * CH, CH)] for j in range(NCH_K)]
        rowidx = [iota16 + jnp.int32(j * CH) for j in range(NCH_K)]
        for d in range(DIM_K):
            cold = jnp.full((16,), d, jnp.int32)
            acc = jnp.zeros((16,), jnp.float32)
            for j in range(NCH_K):
                acc = acc + plsc.load_gather(rows_v, [rowidx[j], cold]) * wvecs[j]
            plsc.store_scatter(outc_v, [iota16, cold],
                               acc * jnp.float32(1.0 / NCH_K))
        pltpu.sync_copy(outc_v, out_ref.at[pl.ds(base + off, CH)])
        return carry

    lax.fori_loop(jnp.int32(0), jnp.int32(NSTEP), step, jnp.int32(0))


_emb_kernel = functools.partial(
    pl.kernel,
    out_type=jax.ShapeDtypeStruct((BATCH_K, DIM_K), jnp.float32),
    mesh=plsc.VectorSubcoreMesh(core_axis_name="c", subcore_axis_name="s"),
    scratch_types=[
        pltpu.VMEM((BPW,), jnp.int32),
        pltpu.VMEM((64,), jnp.int32),
        pltpu.VMEM((G,), jnp.int32),
        pltpu.VMEM((G,), jnp.int32),
        pltpu.VMEM((G, DIM_K), jnp.float32),
        pltpu.VMEM((G,), jnp.float32),
        pltpu.VMEM((CH, DIM_K), jnp.float32),
        pltpu.SemaphoreType.DMA,
        pltpu.SemaphoreType.DMA,
    ],
    compiler_params=pltpu.CompilerParams(needs_layout_passes=False,
                                         use_tc_tiling_on_sc=False),
)(_emb_body)


def kernel(x, table, weights, h0_coeffs, h1_coeffs):
    x32 = x.astype(jnp.int32)
    w_flat = weights.reshape(-1)

    def split(c):
        a, b = c[:, 0], c[:, 1]
        return [(a & 0xFFFF).astype(jnp.int32), (a >> 16).astype(jnp.int32),
                b.astype(jnp.int32)]

    coeffs = jnp.concatenate(split(h0_coeffs) + split(h1_coeffs))
    coeffs = jnp.pad(coeffs, (0, 16))
    return _emb_kernel(x32, table, w_flat, coeffs)
